# baseline (speedup 1.0000x reference)
import functools

import jax
import jax.numpy as jnp
from jax import lax
from jax.experimental import pallas as pl
from jax.experimental.pallas import tpu as pltpu
from jax.experimental.pallas import tpu_sc as plsc

WINDOW = 10
NC = 2
NS = 16
L = 16


def _make_kernel(T, B, D, V):
    NW = NC * NS
    dpw = D // NW
    nacc = WINDOW - 1
    ncf = B // L
    mesh = plsc.VectorSubcoreMesh(core_axis_name="c", subcore_axis_name="s")

    @functools.partial(
        pl.kernel,
        mesh=mesh,
        out_type=jax.ShapeDtypeStruct((D, B), jnp.float32),
        compiler_params=pltpu.CompilerParams(needs_layout_passes=False),
        scratch_types=[
            pltpu.VMEM((B,), jnp.int32),
            pltpu.VMEM((B,), jnp.int32),
            pltpu.VMEM((B,), jnp.float32),
            pltpu.VMEM((B,), jnp.float32),
            pltpu.VMEM((L,), jnp.int32),
            pltpu.VMEM((dpw, L), jnp.int32),
            pltpu.SemaphoreType.DMA,
            pltpu.SemaphoreType.DMA,
            pltpu.SemaphoreType.DMA,
        ],
    )
    def k(data_hbm, atab_hbm, rew_hbm, env_hbm, obs_hbm, toff_hbm, out_hbm,
          env_full, aidx, rww, orow, toff_v, ridx, sem_t, sem_g, sem_o):
        wid = lax.axis_index("s") * NC + lax.axis_index("c")
        d0 = wid * dpw
        lane = lax.iota(jnp.int32, L)

        pltpu.sync_copy(env_hbm, env_full)
        pltpu.sync_copy(toff_hbm, toff_v)

        tv = toff_v[...]
        tvc = jnp.take(tv, jnp.minimum(lane, nacc - 1), axis=0, mode="wrap")
        for j in range(dpw):
            ridx[j, :] = tvc + jnp.int32(d0 + j)

        def phase1(obs_full, rew_full, winbuf):
            pltpu.sync_copy(obs_hbm, obs_full)
            pltpu.sync_copy(rew_hbm, rew_full)
            shift = B.bit_length() - 1

            def win_body(c, carry):
                jv = lane + c * L
                ev = env_full[pl.ds(c * L, L)]
                key = ev * B + jv
                skey, _sv = plsc.sort_key_val(key, jv)
                senv = lax.shift_right_logical(skey, shift)
                sj = lax.bitwise_and(skey, B - 1)
                nxt = jnp.take(senv, jnp.minimum(lane + 1, L - 1), axis=0,
                               mode="wrap")
                m = jnp.logical_or(senv != nxt, lane == L - 1)
                plsc.store_scatter(winbuf, [senv], sj, mask=m)
                return carry

            lax.fori_loop(0, ncf, win_body, None)

            @plsc.parallel_loop(0, ncf, unroll=8)
            def res_body(c):
                sl = pl.ds(c * L, L)
                wv = plsc.load_gather(winbuf, [env_full[sl]])
                aidx[sl] = plsc.load_gather(obs_full, [wv])
                rww[sl] = plsc.load_gather(rew_full, [wv])

        inv = jnp.float32(1.0 / WINDOW)
        out_cps = []

        for j in range(dpw):
            def phase2(arow):
                cp = pltpu.make_async_copy(atab_hbm.at[d0 + j], arow, sem_t)
                cp.start()
                if j == 0:
                    pl.run_scoped(phase1,
                                  pltpu.VMEM((B,), jnp.int32),
                                  pltpu.VMEM((B,), jnp.float32),
                                  pltpu.VMEM((B,), jnp.int32))
                else:
                    out_cps[-1].wait()
                cp.wait()

                @plsc.parallel_loop(0, ncf, unroll=8)
                def a_body(c):
                    sl = pl.ds(c * L, L)
                    g = plsc.load_gather(arow, [aidx[sl]])
                    orow[sl] = g * rww[sl]

            pl.run_scoped(phase2, pltpu.VMEM((V,), jnp.float32))

            def phase3(dstage, wsum):
                pltpu.async_copy(data_hbm.at[ridx.at[j]], dstage, sem_g).wait()

                @plsc.parallel_loop(0, ncf, unroll=8)
                def s_body(c):
                    sl = pl.ds(c * L, L)
                    acc = dstage[0, sl]
                    for t in range(1, nacc):
                        acc = acc + dstage[t, sl]
                    wsum[sl] = acc

                @plsc.parallel_loop(0, ncf, unroll=8)
                def o_body(c):
                    sl = pl.ds(c * L, L)
                    g = plsc.load_gather(wsum, [env_full[sl]])
                    orow[sl] = (g + orow[sl]) * inv

            pl.run_scoped(phase3,
                          pltpu.VMEM((L, B), jnp.float32),
                          pltpu.VMEM((B,), jnp.float32))
            ocp = pltpu.make_async_copy(orow, out_hbm.at[d0 + j], sem_o)
            ocp.start()
            out_cps.append(ocp)

        out_cps[-1].wait()

    return k


def kernel(data, action_table, rew, env_id, obs_next_idx, length):
    T, B, D = data.shape
    V = action_table.shape[0]
    data_t = jnp.transpose(data, (0, 2, 1)).reshape(T * D, B)
    atab_t = action_table.T
    start = jnp.int32(length) - WINDOW
    toff = (start + lax.iota(jnp.int32, L)) * D
    out_t = _make_kernel(T, B, D, V)(
        data_t, atab_t, rew, env_id, obs_next_idx, toff)
    return out_t.T


# speedup vs baseline: 1.0037x; 1.0037x over previous
import functools

import jax
import jax.numpy as jnp
from jax import lax
from jax.experimental import pallas as pl
from jax.experimental.pallas import tpu as pltpu
from jax.experimental.pallas import tpu_sc as plsc

WINDOW = 10
NC = 2
NS = 16
L = 16


def _make_kernel(T, B, D, V):
    NW = NC * NS
    dpw = D // NW
    nacc = WINDOW - 1
    ncf = B // L
    mesh = plsc.VectorSubcoreMesh(core_axis_name="c", subcore_axis_name="s")

    @functools.partial(
        pl.kernel,
        mesh=mesh,
        out_type=jax.ShapeDtypeStruct((D, B), jnp.float32),
        compiler_params=pltpu.CompilerParams(needs_layout_passes=False),
        scratch_types=[
            pltpu.VMEM((B,), jnp.int32),
            pltpu.VMEM((B,), jnp.int32),
            pltpu.VMEM((B,), jnp.float32),
            pltpu.VMEM((B,), jnp.float32),
            pltpu.VMEM((L,), jnp.int32),
            pltpu.VMEM((dpw, L), jnp.int32),
            pltpu.SemaphoreType.DMA,
            pltpu.SemaphoreType.DMA,
            pltpu.SemaphoreType.DMA,
        ],
    )
    def k(data_hbm, atab_hbm, rew_hbm, env_hbm, obs_hbm, toff_hbm, out_hbm,
          env_full, aidx, rww, orow, toff_v, ridx, sem_t, sem_g, sem_o):
        wid = lax.axis_index("s") * NC + lax.axis_index("c")
        d0 = wid * dpw
        lane = lax.iota(jnp.int32, L)

        pltpu.sync_copy(env_hbm, env_full)
        pltpu.sync_copy(toff_hbm, toff_v)

        tv = toff_v[...]
        tvc = jnp.take(tv, jnp.minimum(lane, nacc - 1), axis=0, mode="wrap")
        for j in range(dpw):
            ridx[j, :] = tvc + jnp.int32(d0 + j)

        def phase1(obs_full, rew_full, winbuf):
            pltpu.sync_copy(obs_hbm, obs_full)
            pltpu.sync_copy(rew_hbm, rew_full)
            shift = B.bit_length() - 1

            def win_body(c, carry):
                jv = lane + c * L
                ev = env_full[pl.ds(c * L, L)]
                key = ev * B + jv
                skey, _sv = plsc.sort_key_val(key, jv)
                senv = lax.shift_right_logical(skey, shift)
                sj = lax.bitwise_and(skey, B - 1)
                nxt = jnp.take(senv, jnp.minimum(lane + 1, L - 1), axis=0,
                               mode="wrap")
                m = jnp.logical_or(senv != nxt, lane == L - 1)
                plsc.store_scatter(winbuf, [senv], sj, mask=m)
                return carry

            lax.fori_loop(0, ncf, win_body, None)

            @plsc.parallel_loop(0, ncf, unroll=4)
            def res_body(c):
                sl = pl.ds(c * L, L)
                wv = plsc.load_gather(winbuf, [env_full[sl]])
                aidx[sl] = plsc.load_gather(obs_full, [wv])
                rww[sl] = plsc.load_gather(rew_full, [wv])

        inv = jnp.float32(1.0 / WINDOW)
        out_cps = []

        for j in range(dpw):
            def phase2(arow):
                cp = pltpu.make_async_copy(atab_hbm.at[d0 + j], arow, sem_t)
                cp.start()
                if j == 0:
                    pl.run_scoped(phase1,
                                  pltpu.VMEM((B,), jnp.int32),
                                  pltpu.VMEM((B,), jnp.float32),
                                  pltpu.VMEM((B,), jnp.int32))
                else:
                    out_cps[-1].wait()
                cp.wait()

                @plsc.parallel_loop(0, ncf, unroll=4)
                def a_body(c):
                    sl = pl.ds(c * L, L)
                    g = plsc.load_gather(arow, [aidx[sl]])
                    orow[sl] = g * rww[sl]

            pl.run_scoped(phase2, pltpu.VMEM((V,), jnp.float32))

            def phase3(dstage, wsum):
                pltpu.async_copy(data_hbm.at[ridx.at[j]], dstage, sem_g).wait()

                @plsc.parallel_loop(0, ncf, unroll=4)
                def s_body(c):
                    sl = pl.ds(c * L, L)
                    acc = dstage[0, sl]
                    for t in range(1, nacc):
                        acc = acc + dstage[t, sl]
                    wsum[sl] = acc

                @plsc.parallel_loop(0, ncf, unroll=4)
                def o_body(c):
                    sl = pl.ds(c * L, L)
                    g = plsc.load_gather(wsum, [env_full[sl]])
                    orow[sl] = (g + orow[sl]) * inv

            pl.run_scoped(phase3,
                          pltpu.VMEM((L, B), jnp.float32),
                          pltpu.VMEM((B,), jnp.float32))
            ocp = pltpu.make_async_copy(orow, out_hbm.at[d0 + j], sem_o)
            ocp.start()
            out_cps.append(ocp)

        out_cps[-1].wait()

    return k


def kernel(data, action_table, rew, env_id, obs_next_idx, length):
    T, B, D = data.shape
    V = action_table.shape[0]
    data_t = jnp.transpose(data, (0, 2, 1)).reshape(T * D, B)
    atab_t = action_table.T
    start = jnp.int32(length) - WINDOW
    toff = (start + lax.iota(jnp.int32, L)) * D
    out_t = _make_kernel(T, B, D, V)(
        data_t, atab_t, rew, env_id, obs_next_idx, toff)
    return out_t.T


# speedup vs baseline: 1.1548x; 1.1505x over previous
import functools

import jax
import jax.numpy as jnp
from jax import lax
from jax.experimental import pallas as pl
from jax.experimental.pallas import tpu as pltpu
from jax.experimental.pallas import tpu_sc as plsc

WINDOW = 10
NC = 2
NS = 16
L = 16


def _make_kernel(T, B, D, V):
    NW = NC * NS
    dpw = D // NW
    nacc = WINDOW - 1
    ncf = B // L
    mesh = plsc.VectorSubcoreMesh(core_axis_name="c", subcore_axis_name="s")

    @functools.partial(
        pl.kernel,
        mesh=mesh,
        out_type=jax.ShapeDtypeStruct((D, B), jnp.float32),
        compiler_params=pltpu.CompilerParams(needs_layout_passes=False),
        scratch_types=[
            pltpu.VMEM((B,), jnp.int32),
            pltpu.VMEM((B,), jnp.int32),
            pltpu.VMEM((B,), jnp.float32),
            pltpu.VMEM((B,), jnp.float32),
            pltpu.VMEM((B,), jnp.float32),
            pltpu.VMEM((L,), jnp.int32),
            pltpu.VMEM((24,), jnp.int32),
            pltpu.SemaphoreType.DMA,
            pltpu.SemaphoreType.DMA,
            pltpu.SemaphoreType.DMA,
        ],
    )
    def k(data_hbm, atab_hbm, rew_hbm, env_hbm, obs_hbm, toff_hbm, out_hbm,
          env_full, aidx, rww, orow_a, orow_b, toff_v, ridx24,
          sem_t, sem_g, sem_o):
        wid = lax.axis_index("s") * NC + lax.axis_index("c")
        d0 = wid * dpw
        lane = lax.iota(jnp.int32, L)
        orows = [orow_a, orow_b]

        pltpu.sync_copy(env_hbm, env_full)
        pltpu.sync_copy(toff_hbm, toff_v)

        tv = toff_v[...]
        s0 = lane
        v0 = jnp.take(tv, jnp.where(s0 < nacc, s0, s0 - nacc), axis=0,
                      mode="wrap") + jnp.where(s0 < nacc, d0, d0 + 1)
        s1 = lane + 8
        v1 = (jnp.take(tv,
                       jnp.where(s1 < nacc, s1,
                                 jnp.where(s1 < 2 * nacc, s1 - nacc, 0)),
                       axis=0, mode="wrap")
              + jnp.where(s1 < nacc, d0,
                          jnp.where(s1 < 2 * nacc, d0 + 1, d0)))
        ridx24[pl.ds(0, L)] = v0
        ridx24[pl.ds(8, L)] = v1

        def win_phase(winbuf):
            shift = B.bit_length() - 1

            def win_body(c, carry):
                jv = lane + c * L
                ev = env_full[pl.ds(c * L, L)]
                key = ev * B + jv
                skey, _sv = plsc.sort_key_val(key, jv)
                senv = lax.shift_right_logical(skey, shift)
                sj = lax.bitwise_and(skey, B - 1)
                nxt = jnp.take(senv, jnp.minimum(lane + 1, L - 1), axis=0,
                               mode="wrap")
                m = jnp.logical_or(senv != nxt, lane == L - 1)
                plsc.store_scatter(winbuf, [senv], sj, mask=m)
                return carry

            lax.fori_loop(0, ncf, win_body, None)

            @plsc.parallel_loop(0, ncf, unroll=4)
            def wf_body(c):
                sl = pl.ds(c * L, L)
                aidx[sl] = plsc.load_gather(winbuf, [env_full[sl]])

        def res_phase(obs_full, rew_full):
            pltpu.sync_copy(obs_hbm, obs_full)
            pltpu.sync_copy(rew_hbm, rew_full)

            @plsc.parallel_loop(0, ncf, unroll=4)
            def res_body(c):
                sl = pl.ds(c * L, L)
                wv = aidx[sl]
                rww[sl] = plsc.load_gather(rew_full, [wv])
                aidx[sl] = plsc.load_gather(obs_full, [wv])

        inv = jnp.float32(1.0 / WINDOW)

        for j in range(dpw):
            def phase2(arow):
                cp = pltpu.make_async_copy(atab_hbm.at[d0 + j], arow, sem_t)
                cp.start()
                if j == 0:
                    pl.run_scoped(win_phase, pltpu.VMEM((B,), jnp.int32))
                    pl.run_scoped(res_phase,
                                  pltpu.VMEM((B,), jnp.int32),
                                  pltpu.VMEM((B,), jnp.float32))
                cp.wait()
                orow = orows[j]

                @plsc.parallel_loop(0, ncf, unroll=4)
                def a_body(c):
                    sl = pl.ds(c * L, L)
                    g = plsc.load_gather(arow, [aidx[sl]])
                    orow[sl] = g * rww[sl]

            pl.run_scoped(phase2, pltpu.VMEM((V,), jnp.float32))

        def phase3(dstage, wsum):
            pltpu.async_copy(data_hbm.at[ridx24], dstage, sem_g).wait()
            out_cps = []
            for j in range(dpw):
                orow = orows[j]

                @plsc.parallel_loop(0, ncf, unroll=4)
                def s_body(c):
                    sl = pl.ds(c * L, L)
                    acc = dstage[j * nacc, sl]
                    for t in range(1, nacc):
                        acc = acc + dstage[j * nacc + t, sl]
                    wsum[sl] = acc

                @plsc.parallel_loop(0, ncf, unroll=4)
                def o_body(c):
                    sl = pl.ds(c * L, L)
                    g = plsc.load_gather(wsum, [env_full[sl]])
                    orow[sl] = (g + orow[sl]) * inv

                ocp = pltpu.make_async_copy(orow, out_hbm.at[d0 + j], sem_o)
                ocp.start()
                out_cps.append(ocp)
            for ocp in out_cps:
                ocp.wait()

        pl.run_scoped(phase3,
                      pltpu.VMEM((24, B), jnp.float32),
                      pltpu.VMEM((B,), jnp.float32))

    return k


def kernel(data, action_table, rew, env_id, obs_next_idx, length):
    T, B, D = data.shape
    V = action_table.shape[0]
    data_t = jnp.transpose(data, (0, 2, 1)).reshape(T * D, B)
    atab_t = action_table.T
    start = jnp.int32(length) - WINDOW
    toff = (start + lax.iota(jnp.int32, L)) * D
    out_t = _make_kernel(T, B, D, V)(
        data_t, atab_t, rew, env_id, obs_next_idx, toff)
    return out_t.T
